# baseline (speedup 1.0000x reference)
import functools

import jax
import jax.numpy as jnp
from jax import lax
from jax.experimental import pallas as pl
from jax.experimental.pallas import tpu as pltpu
from jax.experimental.pallas import tpu_sc as plsc

B = 4096
NF = 26
V = 100000
D = 16
ND = 13
H1, H2 = 256, 128
BNF = B * NF
NC, NS = 2, 16
NW = NC * NS
PER_W = BNF // NW
CHUNK = 128
NCHUNK = PER_W // CHUNK


G = 2
NG = NCHUNK // G


def _sc_gather(emb_table, lin16, idx3d, idxh3d):
    mesh = plsc.VectorSubcoreMesh(core_axis_name="c", subcore_axis_name="s")

    @functools.partial(
        pl.kernel,
        mesh=mesh,
        compiler_params=pltpu.CompilerParams(use_tc_tiling_on_sc=False),
        out_type=[
            jax.ShapeDtypeStruct((BNF, D), jnp.float32),
            jax.ShapeDtypeStruct((BNF, D), jnp.float32),
        ],
        scratch_types=[
            pltpu.VMEM((NCHUNK, CHUNK), jnp.int32),
            pltpu.VMEM((NCHUNK, CHUNK), jnp.int32),
            pltpu.VMEM((PER_W, D), jnp.float32),
            pltpu.VMEM((PER_W, D), jnp.float32),
            pltpu.SemaphoreType.DMA,
            pltpu.SemaphoreType.DMA,
        ],
    )
    def gather_k(emb_hbm, lin_hbm, idx_hbm, idxh_hbm, e_out, l_out,
                 idx_v, idxh_v, rows_v, lrows_v, sem_e, sem_l):
        wid = lax.axis_index("s") * NC + lax.axis_index("c")
        base = wid * PER_W
        pltpu.sync_copy(idx_hbm.at[wid], idx_v)
        pltpu.sync_copy(idxh_hbm.at[wid], idxh_v)

        def fire(g):
            for b in range(G):
                k = g * G + b
                pltpu.async_copy(emb_hbm.at[idx_v.at[k]],
                                 rows_v.at[pl.ds(k * CHUNK, CHUNK)], sem_e)
                pltpu.async_copy(lin_hbm.at[idxh_v.at[k]],
                                 lrows_v.at[pl.ds(k * CHUNK, CHUNK)], sem_l)

        def drain(g):
            off = g * G * CHUNK
            pltpu.make_async_copy(emb_hbm.at[pl.ds(0, G * CHUNK)],
                                  rows_v.at[pl.ds(off, G * CHUNK)],
                                  sem_e).wait()
            pltpu.make_async_copy(lin_hbm.at[pl.ds(0, G * CHUNK)],
                                  lrows_v.at[pl.ds(off, G * CHUNK)],
                                  sem_l).wait()

        fire(0)

        @pl.loop(1, NG)
        def _(g):
            fire(g)
            drain(g - 1)

        drain(NG - 1)

        pltpu.sync_copy(rows_v, e_out.at[pl.ds(base, PER_W)])
        pltpu.sync_copy(lrows_v, l_out.at[pl.ds(base, PER_W)])

    return gather_k(emb_table, lin16, idx3d, idxh3d)


BC = 512
NCH = B // BC

_HI = jax.lax.Precision.HIGHEST


def _dot(a, b):
    return jnp.dot(a, b, precision=_HI, preferred_element_type=jnp.float32)


def _tc_body(dense_ref, e_ref, l16_ref, lo_ref, R_ref, colv_ref, S_ref,
             Wd_ref, W1a_ref, W1b_ref,
             b1_ref, g1_ref, be1_ref, W2_ref, b2_ref, g2_ref, be2_ref,
             Wo_ref, bias_ref, out_ref,
             z1_scr, z2_scr, a1s, a1q, a2s, a2q):
    p = pl.program_id(0)
    i = pl.program_id(1)

    @pl.when((p == 0) & (i == 0))
    def _():
        a1s[...] = jnp.zeros_like(a1s)
        a1q[...] = jnp.zeros_like(a1q)
        a2s[...] = jnp.zeros_like(a2s)
        a2q[...] = jnp.zeros_like(a2q)

    @pl.when(p == 0)
    def _():
        z1 = (_dot(dense_ref[...], W1a_ref[...])
              + _dot(e_ref[...], W1b_ref[...]) + b1_ref[...])
        z1_scr[pl.ds(i * BC, BC), :] = z1
        a1s[...] += jnp.sum(z1, axis=0, keepdims=True)
        a1q[...] += jnp.sum(z1 * z1, axis=0, keepdims=True)
        out_ref[...] = jnp.zeros_like(out_ref)

    @pl.when(p == 1)
    def _():
        m1 = a1s[...] * (1.0 / B)
        v1 = a1q[...] * (1.0 / B) - m1 * m1
        z1 = z1_scr[pl.ds(i * BC, BC), :]
        h1 = jnp.maximum((z1 - m1) * lax.rsqrt(v1 + 1e-5) * g1_ref[...]
                         + be1_ref[...], 0.0)
        z2 = _dot(h1, W2_ref[...]) + b2_ref[...]
        z2_scr[pl.ds(i * BC, BC), :] = z2
        a2s[...] += jnp.sum(z2, axis=0, keepdims=True)
        a2q[...] += jnp.sum(z2 * z2, axis=0, keepdims=True)
        out_ref[...] = jnp.zeros_like(out_ref)

    @pl.when(p == 2)
    def _():
        dense = dense_ref[...]
        e = e_ref[...]
        lo_exp = _dot(lo_ref[...], R_ref[...])
        sel = jnp.where(jnp.abs(colv_ref[...] - lo_exp) < 0.5, l16_ref[...],
                        0.0)
        linear = (_dot(dense, Wd_ref[...])
                  + jnp.sum(sel, axis=1, keepdims=True))
        s = _dot(e, S_ref[...])
        ss = _dot(e * e, S_ref[...])
        fm = 0.5 * jnp.sum(s * s - ss, axis=1, keepdims=True)

        m2 = a2s[...] * (1.0 / B)
        v2 = a2q[...] * (1.0 / B) - m2 * m2
        z2 = z2_scr[pl.ds(i * BC, BC), :]
        h2 = jnp.maximum((z2 - m2) * lax.rsqrt(v2 + 1e-5) * g2_ref[...]
                         + be2_ref[...], 0.0)
        deep = _dot(h2, Wo_ref[...])
        out_ref[...] = linear + fm + deep + bias_ref[...]


def _tc_forward(dense_features, e_flat, l16_flat, lo_f32, W_dense, b_dense,
                W1, b1, g1, be1, W2, b2, g2, be2, W_out, b_out,
                interpret=False):
    S = jnp.tile(jnp.eye(D, dtype=jnp.float32), (NF, 1))
    R = jnp.kron(jnp.eye(NF, dtype=jnp.float32),
                 jnp.ones((1, D), dtype=jnp.float32))
    colv = jnp.tile(jnp.arange(D, dtype=jnp.float32), NF).reshape(1, NF * D)
    bias = (b_dense + b_out).reshape(1, 1).astype(jnp.float32)

    chunk = lambda p, i: (i, 0)
    whole = lambda p, i: (0, 0)
    out = pl.pallas_call(
        _tc_body,
        grid=(3, NCH),
        in_specs=[
            pl.BlockSpec((BC, ND), chunk),
            pl.BlockSpec((BC, NF * D), chunk),
            pl.BlockSpec((BC, NF * D), chunk),
            pl.BlockSpec((BC, NF), chunk),
            pl.BlockSpec((NF, NF * D), whole),
            pl.BlockSpec((1, NF * D), whole),
            pl.BlockSpec((NF * D, D), whole),
            pl.BlockSpec((ND, 1), whole),
            pl.BlockSpec((ND, H1), whole),
            pl.BlockSpec((NF * D, H1), whole),
            pl.BlockSpec((1, H1), whole),
            pl.BlockSpec((1, H1), whole),
            pl.BlockSpec((1, H1), whole),
            pl.BlockSpec((H1, H2), whole),
            pl.BlockSpec((1, H2), whole),
            pl.BlockSpec((1, H2), whole),
            pl.BlockSpec((1, H2), whole),
            pl.BlockSpec((H2, 1), whole),
            pl.BlockSpec((1, 1), whole),
        ],
        out_specs=pl.BlockSpec((BC, 1), chunk),
        scratch_shapes=[
            pltpu.VMEM((B, H1), jnp.float32),
            pltpu.VMEM((B, H2), jnp.float32),
            pltpu.VMEM((1, H1), jnp.float32),
            pltpu.VMEM((1, H1), jnp.float32),
            pltpu.VMEM((1, H2), jnp.float32),
            pltpu.VMEM((1, H2), jnp.float32),
        ],
        out_shape=jax.ShapeDtypeStruct((B, 1), jnp.float32),
        interpret=interpret,
    )(dense_features, e_flat, l16_flat, lo_f32, R, colv, S,
      W_dense.reshape(ND, 1), W1[:ND], W1[ND:], b1.reshape(1, H1),
      g1.reshape(1, H1), be1.reshape(1, H1), W2, b2.reshape(1, H2),
      g2.reshape(1, H2), be2.reshape(1, H2), W_out.reshape(H2, 1), bias)
    return out.reshape(B)


def kernel(dense_features, sparse_features, emb_table, lin_table, W_dense,
           b_dense, W1, b1, g1, be1, W2, b2, g2, be2, W_out, b_out):
    offsets = sparse_features + jnp.arange(NF, dtype=jnp.int32)[None, :] * V
    idx3d = offsets.reshape(NW, NCHUNK, CHUNK)
    idxh3d = (offsets // D).reshape(NW, NCHUNK, CHUNK)
    lin16 = lin_table.reshape(NF * V // D, D)

    e_rows, l_rows = _sc_gather(emb_table, lin16, idx3d, idxh3d)
    e_flat = e_rows.reshape(B, NF * D)
    l16_flat = l_rows.reshape(B, NF * D)
    lo_f32 = (offsets % D).astype(jnp.float32)

    return _tc_forward(dense_features, e_flat, l16_flat, lo_f32, W_dense,
                       b_dense, W1, b1, g1, be1, W2, b2, g2, be2, W_out,
                       b_out)
